# baseline (speedup 1.0000x reference)
import functools

import jax
import jax.numpy as jnp
import numpy as np
from jax import lax
from jax.experimental import pallas as pl
from jax.experimental.pallas import tpu as pltpu
from jax.experimental.pallas import tpu_sc as plsc

LANES = 16


def _pos_encoding_np(length: int, d_model: int) -> np.ndarray:
    position = np.arange(0, length, dtype=np.float32)[:, None]
    i2 = np.arange(0, d_model, step=2).astype(np.float32)
    emb = np.zeros((length, d_model), dtype=np.float32)
    emb[:, 0::2] = np.sin(position / 10000 ** (i2 / d_model))
    emb[:, 1::2] = np.cos(position / 10000 ** (i2 / d_model))
    return emb


@functools.lru_cache(maxsize=None)
def _pos_const(length: int, d_model: int):
    return jnp.asarray(_pos_encoding_np(length, d_model))


def _sc_info():
    try:
        info = plsc.get_sparse_core_info()
        return info.num_cores, info.num_subcores
    except Exception:
        return 2, 16


@functools.lru_cache(maxsize=None)
def _build(B: int, L: int, D: int):
    NC, NS = _sc_info()
    NW = NC * NS
    assert L % NW == 0
    cols = L // NW
    CH = 32
    assert cols % CH == 0
    n_chunks_per_b = cols // CH
    nvec = D // LANES

    mesh = plsc.VectorSubcoreMesh(core_axis_name="c", subcore_axis_name="s")

    @functools.partial(
        pl.kernel,
        mesh=mesh,
        out_type=jax.ShapeDtypeStruct((B, L, D), jnp.float32),
        scratch_types=[
            pltpu.VMEM((B, cols), jnp.int32),
            pltpu.VMEM((cols, D), jnp.float32),
            pltpu.VMEM((CH, D), jnp.float32),
            pltpu.SemaphoreType.DMA,
        ],
    )
    def k(x_hbm, table_hbm, pos_hbm, out_hbm, idx_v, pos_v, buf, sem):
        wid = lax.axis_index("s") * NC + lax.axis_index("c")
        l0 = wid * cols
        for b in range(B):
            pltpu.sync_copy(x_hbm.at[b, pl.ds(l0, cols)], idx_v.at[b])
        pltpu.sync_copy(pos_hbm.at[pl.ds(l0, cols)], pos_v)

        for b in range(B):
            for c in range(n_chunks_per_b):
                off = c * CH
                pltpu.async_copy(
                    table_hbm.at[idx_v.at[b, pl.ds(off, CH)]], buf, sem
                ).wait()

                def row_body(r, _, off=off):
                    for j in range(nvec):
                        sl = pl.ds(j * LANES, LANES)
                        buf[r, sl] = buf[r, sl] + pos_v[off + r, sl]
                    return 0

                lax.fori_loop(0, CH, row_body, 0)
                pltpu.sync_copy(buf, out_hbm.at[b, pl.ds(l0 + off, CH)])

    return k


def kernel(x, table):
    B, L = x.shape
    D = table.shape[1]
    pos = _pos_const(L, D)
    return _build(B, L, D)(x, table, pos)
